# baseline (speedup 1.0000x reference)
import jax
import jax.numpy as jnp
from jax import lax
from jax.experimental import pallas as pl
from jax.experimental.pallas import tpu as pltpu
from jax.experimental.pallas import tpu_sc as plsc

N_NODES = 10000
D_IN = 128
D_HID = 16
D_OUT = 128
N_EDGES = 320000

N_PAD = 10240
HALF = N_PAD // 2
E_TOT = N_EDGES + N_NODES
CH = 128
N_WORKERS = 32
CPW = -(-E_TOT // (N_WORKERS * CH))
E_PAD = N_WORKERS * CPW * CH
SUP = 9
CAPT = CPW * CH
CAPW = CAPT + CH
ROWS_PT = HALF // 16

_SC_MESH = plsc.VectorSubcoreMesh(core_axis_name="c", subcore_axis_name="s")
_SC_PARAMS = pltpu.CompilerParams(
    needs_layout_passes=False, use_tc_tiling_on_sc=False)


def _tc_dense_a(x_ref, w_ref, asrc_ref, adst_ref, h_ref, as_ref, ad_ref):
    h = jnp.dot(x_ref[...], w_ref[...], preferred_element_type=jnp.float32)
    h_ref[...] = h
    as_ref[...] = jnp.sum(h * asrc_ref[...], axis=1)
    ad_ref[...] = jnp.sum(h * adst_ref[...], axis=1)


def _tc_mid(s_ref, dn_ref, b1_ref, w2_ref, asrc_ref, adst_ref,
            h2_ref, as_ref, ad_ref):
    den = jnp.sum(dn_ref[...], axis=0)[:, None] + 1e-16
    hin = jnp.maximum(s_ref[...] / den + b1_ref[...], 0.0)
    h2 = jnp.dot(hin, w2_ref[...], preferred_element_type=jnp.float32)
    h2_ref[...] = h2
    as_ref[...] = jnp.sum(h2 * asrc_ref[...], axis=1)
    ad_ref[...] = jnp.sum(h2 * adst_ref[...], axis=1)


def _tc_final(s_ref, dn_ref, b2_ref, out_ref):
    den = jnp.sum(dn_ref[...], axis=0)[:, None] + 1e-16
    out_ref[...] = s_ref[...] / den + b2_ref[...]


def _sc_w_body(src_hbm, dst_hbm, as_hbm, ad_hbm, li_out, cnt_out, dn_out,
               as_v, ad_v, dn_v, sd_v, cl_v, ch_v, cnt_v):
    c = lax.axis_index("c")
    s = lax.axis_index("s")
    wid = c * 16 + s
    pltpu.sync_copy(as_hbm, as_v)
    pltpu.sync_copy(ad_hbm, ad_v)

    zv = jnp.zeros((16,), jnp.float32)

    def zdn(i, carry):
        dn_v[pl.ds(i * 16, 16)] = zv
        return carry

    lax.fori_loop(0, N_PAD // 16, zdn, 0)

    def sup(s9, carry):
        nlo, nhi = carry
        ebase = wid * CAPT + s9 * (SUP * CH)
        pltpu.sync_copy(src_hbm.at[pl.ds(ebase, SUP * CH)], sd_v.at[0])
        pltpu.sync_copy(dst_hbm.at[pl.ds(ebase, SUP * CH)], sd_v.at[1])

        def chunk(k, carry2):
            nlo2, nhi2 = carry2
            for j in range(CH // 16):
                sv = sd_v[0, pl.ds(k * CH + j * 16, 16)]
                dv = sd_v[1, pl.ds(k * CH + j * 16, 16)]
                v = (plsc.load_gather(as_v, [sv])
                     + plsc.load_gather(ad_v, [dv]))
                v = jnp.where(v >= 0, v, 0.2 * v)
                w = jnp.exp(v)
                plsc.addupdate_scatter(dn_v, [dv], w)
                wb = plsc.bitcast(w, jnp.int32)
                mlo = dv < HALF
                mhi = jnp.logical_not(mlo)
                plsc.store_compressed(cl_v.at[0, pl.ds(nlo2, 16)], sv, mask=mlo)
                plsc.store_compressed(cl_v.at[1, pl.ds(nlo2, 16)], dv, mask=mlo)
                plsc.store_compressed(cl_v.at[2, pl.ds(nlo2, 16)], wb, mask=mlo)
                plsc.store_compressed(ch_v.at[0, pl.ds(nhi2, 16)], sv, mask=mhi)
                plsc.store_compressed(ch_v.at[1, pl.ds(nhi2, 16)],
                                      dv - HALF, mask=mhi)
                plsc.store_compressed(ch_v.at[2, pl.ds(nhi2, 16)], wb, mask=mhi)
                clo = jnp.sum(mlo.astype(jnp.int32))
                nlo2 = nlo2 + clo
                nhi2 = nhi2 + (16 - clo)
            return nlo2, nhi2

        return lax.fori_loop(0, SUP, chunk, (nlo, nhi))

    nlo, nhi = lax.fori_loop(0, CPW // SUP, sup,
                             (jnp.int32(0), jnp.int32(0)))

    zvi = jnp.zeros((16,), jnp.int32)
    for a in range(3):
        for k in range(CH // 16):
            cl_v[a, pl.ds(nlo + k * 16, 16)] = zvi
            ch_v[a, pl.ds(nhi + k * 16, 16)] = zvi
    cnt_v[0, :] = jnp.full((16,), nlo, jnp.int32)
    cnt_v[1, :] = jnp.full((16,), nhi, jnp.int32)

    pltpu.sync_copy(cl_v, li_out.at[0, wid])
    pltpu.sync_copy(ch_v, li_out.at[1, wid])
    pltpu.sync_copy(cnt_v, cnt_out.at[wid])
    pltpu.sync_copy(dn_v, dn_out.at[wid])


_sc_w = pl.kernel(
    _sc_w_body,
    out_type=(
        jax.ShapeDtypeStruct((2, N_WORKERS, 3, CAPW), jnp.int32),
        jax.ShapeDtypeStruct((N_WORKERS, 2, 16), jnp.int32),
        jax.ShapeDtypeStruct((N_WORKERS, N_PAD), jnp.float32),
    ),
    mesh=_SC_MESH,
    compiler_params=_SC_PARAMS,
    scratch_types=[
        pltpu.VMEM((N_PAD,), jnp.float32),
        pltpu.VMEM((N_PAD,), jnp.float32),
        pltpu.VMEM((N_PAD,), jnp.float32),
        pltpu.VMEM((2, SUP * CH), jnp.int32),
        pltpu.VMEM((3, CAPW), jnp.int32),
        pltpu.VMEM((3, CAPW), jnp.int32),
        pltpu.VMEM((2, 16), jnp.int32),
    ],
)


def _make_sc_agg(C):
    def _scale(big_v, wring_v, ebuf_v, bg, g3):
        for j in range(CH // 16):
            wring_v[pl.ds(j * 16, 16)] = plsc.bitcast(
                ebuf_v[g3, 2, pl.ds(j * 16, 16)], jnp.float32)

        def grp(j2, carry):
            for l in range(16):
                e = j2 * 16 + l
                ws = plsc.load_gather(
                    wring_v, [jnp.full((16,), e, jnp.int32)])
                for k in range(C // 16):
                    big_v[bg, e, pl.ds(k * 16, 16)] = (
                        big_v[bg, e, pl.ds(k * 16, 16)] * ws)
            return carry

        lax.fori_loop(0, CH // 16, grp, 0)

    def body(hp_hbm, li_hbm, cnt_hbm, s_out,
             ebuf_v, big_v, wring_v, cnt_v, s_sh, sem_g, sem_s, sem_i):
        c = lax.axis_index("c")
        s = lax.axis_index("s")
        r0 = s * ROWS_PT

        zv = jnp.zeros((16,), jnp.float32)

        def zrow(i, carry):
            for j in range(C // 16):
                big_v[1, i, pl.ds(j * 16, 16)] = zv
            return carry

        lax.fori_loop(0, CH, zrow, 0)
        pltpu.sync_copy(big_v.at[1], s_sh.at[pl.ds(r0, CH)])
        pltpu.sync_copy(big_v.at[1], s_sh.at[pl.ds(r0 + CH, CH)])
        pltpu.sync_copy(big_v.at[1, pl.ds(0, ROWS_PT - 2 * CH)],
                        s_sh.at[pl.ds(r0 + 2 * CH, ROWS_PT - 2 * CH)])
        plsc.subcore_barrier()

        def run_list(wtile):
            pltpu.sync_copy(cnt_hbm.at[wtile, c], cnt_v)
            nch = (jnp.max(cnt_v[...]) + CH - 1) // CH
            nch = jnp.maximum(nch, 1)

            def tr_start(slot, g):
                for a in range(3):
                    pltpu.async_copy(
                        li_hbm.at[c, wtile, a, pl.ds(g * CH, CH)],
                        ebuf_v.at[slot, a], sem_i)

            def tr_wait(slot, g):
                for a in range(3):
                    pltpu.make_async_copy(
                        li_hbm.at[c, wtile, a, pl.ds(g * CH, CH)],
                        ebuf_v.at[slot, a], sem_i).wait()

            for a in range(3):
                pltpu.sync_copy(li_hbm.at[c, wtile, a, pl.ds(0, CH)],
                                ebuf_v.at[0, a])
            pltpu.async_copy(hp_hbm.at[ebuf_v.at[0, 0]], big_v.at[0], sem_g)
            tr_start(1, jnp.minimum(1, nch - 1))

            pltpu.make_async_copy(
                hp_hbm.at[ebuf_v.at[0, 0]], big_v.at[0], sem_g).wait()
            tr_start(2, jnp.minimum(2, nch - 1))
            tr_wait(1, jnp.minimum(1, nch - 1))
            pltpu.async_copy(hp_hbm.at[ebuf_v.at[1, 0]], big_v.at[1], sem_g)
            _scale(big_v, wring_v, ebuf_v, 0, 0)
            pltpu.async_copy(big_v.at[0], s_sh.at[ebuf_v.at[0, 1]], sem_s,
                             add=True)

            def iter_g(g, carry):
                bg = lax.rem(g, 2)
                nbg = 1 - bg
                g3 = lax.rem(g, 3)
                gn3 = lax.rem(g + 1, 3)
                gp3 = lax.rem(g + 2, 3)
                pltpu.make_async_copy(
                    hp_hbm.at[ebuf_v.at[g3, 0]], big_v.at[bg], sem_g).wait()
                pltpu.make_async_copy(
                    big_v.at[nbg], s_sh.at[ebuf_v.at[gp3, 1]], sem_s).wait()
                tr_start(gp3, jnp.minimum(g + 2, nch - 1))
                tr_wait(gn3, jnp.minimum(g + 1, nch - 1))
                pltpu.async_copy(
                    hp_hbm.at[ebuf_v.at[gn3, 0]], big_v.at[nbg], sem_g)
                _scale(big_v, wring_v, ebuf_v, bg, g3)
                pltpu.async_copy(big_v.at[bg], s_sh.at[ebuf_v.at[g3, 1]],
                                 sem_s, add=True)
                return carry

            lax.fori_loop(1, nch, iter_g, 0)

            bl = lax.rem(nch - 1, 2)
            pltpu.make_async_copy(
                big_v.at[bl],
                s_sh.at[ebuf_v.at[lax.rem(nch - 1, 3), 1]], sem_s).wait()
            pltpu.make_async_copy(
                hp_hbm.at[ebuf_v.at[lax.rem(nch, 3), 0]],
                big_v.at[1 - bl], sem_g).wait()
            tr_wait(lax.rem(nch + 1, 3), nch - 1)

        run_list(s)
        run_list(s + 16)
        plsc.subcore_barrier()
        pltpu.sync_copy(s_sh.at[pl.ds(r0, ROWS_PT)],
                        s_out.at[pl.ds(c * HALF + r0, ROWS_PT)])

    return pl.kernel(
        body,
        out_type=jax.ShapeDtypeStruct((N_PAD, C), jnp.float32),
        mesh=_SC_MESH,
        compiler_params=_SC_PARAMS,
        scratch_types=[
            pltpu.VMEM((3, 3, CH), jnp.int32),
            pltpu.VMEM((2, CH, C), jnp.float32),
            pltpu.VMEM((CH,), jnp.float32),
            pltpu.VMEM((16,), jnp.int32),
            pltpu.VMEM_SHARED((HALF, C), jnp.float32),
            pltpu.SemaphoreType.DMA,
            pltpu.SemaphoreType.DMA,
            pltpu.SemaphoreType.DMA,
        ],
    )


_sc_agg_1 = _make_sc_agg(D_HID)
_sc_agg_2 = _make_sc_agg(D_OUT)

_dense_a = pl.pallas_call(
    _tc_dense_a,
    out_shape=(
        jax.ShapeDtypeStruct((N_PAD, D_HID), jnp.float32),
        jax.ShapeDtypeStruct((N_PAD,), jnp.float32),
        jax.ShapeDtypeStruct((N_PAD,), jnp.float32),
    ),
)

_mid = pl.pallas_call(
    _tc_mid,
    out_shape=(
        jax.ShapeDtypeStruct((N_PAD, D_OUT), jnp.float32),
        jax.ShapeDtypeStruct((N_PAD,), jnp.float32),
        jax.ShapeDtypeStruct((N_PAD,), jnp.float32),
    ),
)

_final = pl.pallas_call(
    _tc_final,
    out_shape=jax.ShapeDtypeStruct((N_PAD, D_OUT), jnp.float32),
)


def kernel(x, edge_index, W1, att_src1, att_dst1, b1,
           W2, att_src2, att_dst2, b2):
    x = x.astype(jnp.float32)
    loops = jnp.arange(N_NODES, dtype=jnp.int32)
    padv = jnp.full((E_PAD - E_TOT,), N_NODES, dtype=jnp.int32)
    src = jnp.concatenate([edge_index[0].astype(jnp.int32), loops, padv])
    dst = jnp.concatenate([edge_index[1].astype(jnp.int32), loops, padv])

    x_pad = jnp.zeros((N_PAD, D_IN), jnp.float32).at[:N_NODES].set(x)

    h1, as1, ad1 = _dense_a(x_pad, W1, att_src1.reshape(1, D_HID),
                            att_dst1.reshape(1, D_HID))
    li1, cn1, dn1 = _sc_w(src, dst, as1, ad1)
    s1 = _sc_agg_1(h1, li1, cn1)
    h2, as2, ad2 = _mid(s1, dn1, b1.reshape(1, D_HID), W2,
                        att_src2.reshape(1, D_OUT), att_dst2.reshape(1, D_OUT))
    li2, cn2, dn2 = _sc_w(src, dst, as2, ad2)
    s2 = _sc_agg_2(h2, li2, cn2)
    out = _final(s2, dn2, b2.reshape(1, D_OUT))
    return out[:N_NODES]


# speedup vs baseline: 1.4067x; 1.4067x over previous
import jax
import jax.numpy as jnp
from jax import lax
from jax.experimental import pallas as pl
from jax.experimental.pallas import tpu as pltpu
from jax.experimental.pallas import tpu_sc as plsc

N_NODES = 10000
D_IN = 128
D_HID = 16
D_OUT = 128
N_EDGES = 320000

N_PAD = 10240
E_TOT = N_EDGES + N_NODES
CH = 128
N_WORKERS = 32
CPW = -(-E_TOT // (N_WORKERS * CH))
E_PAD = N_WORKERS * CPW * CH
NCHUNKS = N_WORKERS * CPW
SUP = 9
ROWS_PT = N_PAD // 16

_SC_MESH = plsc.VectorSubcoreMesh(core_axis_name="c", subcore_axis_name="s")
_SC_PARAMS = pltpu.CompilerParams(
    needs_layout_passes=False, use_tc_tiling_on_sc=False)


def _tc_dense_a(x_ref, w_ref, asrc_ref, adst_ref, h_ref, as_ref, ad_ref):
    h = jnp.dot(x_ref[...], w_ref[...], preferred_element_type=jnp.float32)
    h_ref[...] = h
    as_ref[...] = jnp.sum(h * asrc_ref[...], axis=1)
    ad_ref[...] = jnp.sum(h * adst_ref[...], axis=1)


def _tc_mid(s_ref, dn_ref, b1_ref, w2_ref, asrc_ref, adst_ref,
            h2_ref, as_ref, ad_ref):
    s = s_ref[0] + s_ref[1]
    den = jnp.sum(dn_ref[...], axis=0)[:, None] + 1e-16
    hin = jnp.maximum(s / den + b1_ref[...], 0.0)
    h2 = jnp.dot(hin, w2_ref[...], preferred_element_type=jnp.float32)
    h2_ref[...] = h2
    as_ref[...] = jnp.sum(h2 * asrc_ref[...], axis=1)
    ad_ref[...] = jnp.sum(h2 * adst_ref[...], axis=1)


def _tc_final(s_ref, dn_ref, b2_ref, out_ref):
    s = s_ref[0] + s_ref[1]
    den = jnp.sum(dn_ref[...], axis=0)[:, None] + 1e-16
    out_ref[...] = s / den + b2_ref[...]


def _sc_w_body(src_hbm, dst_hbm, as_hbm, ad_hbm, pk_out, dn_out,
               as_v, ad_v, dn_v, sd_v, stage_v):
    c = lax.axis_index("c")
    s = lax.axis_index("s")
    wid = c * 16 + s
    pltpu.sync_copy(as_hbm, as_v)
    pltpu.sync_copy(ad_hbm, ad_v)

    zv = jnp.zeros((16,), jnp.float32)

    def zdn(i, carry):
        dn_v[pl.ds(i * 16, 16)] = zv
        return carry

    lax.fori_loop(0, N_PAD // 16, zdn, 0)

    def sup(s9, carry):
        ebase = wid * (CPW * CH) + s9 * (SUP * CH)
        pltpu.sync_copy(src_hbm.at[pl.ds(ebase, SUP * CH)], sd_v.at[0])
        pltpu.sync_copy(dst_hbm.at[pl.ds(ebase, SUP * CH)], sd_v.at[1])

        def chunk(k, carry2):
            for j in range(CH // 16):
                sv = sd_v[0, pl.ds(k * CH + j * 16, 16)]
                dv = sd_v[1, pl.ds(k * CH + j * 16, 16)]
                v = (plsc.load_gather(as_v, [sv])
                     + plsc.load_gather(ad_v, [dv]))
                v = jnp.where(v >= 0, v, 0.2 * v)
                w = jnp.exp(v)
                plsc.addupdate_scatter(dn_v, [dv], w)
                stage_v[k, 0, pl.ds(j * 16, 16)] = sv
                stage_v[k, 1, pl.ds(j * 16, 16)] = dv
                stage_v[k, 2, pl.ds(j * 16, 16)] = plsc.bitcast(w, jnp.int32)
            return carry2

        lax.fori_loop(0, SUP, chunk, 0)
        pltpu.sync_copy(stage_v, pk_out.at[pl.ds(wid * CPW + s9 * SUP, SUP)])
        return carry

    lax.fori_loop(0, CPW // SUP, sup, 0)
    pltpu.sync_copy(dn_v, dn_out.at[wid])


_sc_w = pl.kernel(
    _sc_w_body,
    out_type=(
        jax.ShapeDtypeStruct((NCHUNKS, 3, CH), jnp.int32),
        jax.ShapeDtypeStruct((N_WORKERS, N_PAD), jnp.float32),
    ),
    mesh=_SC_MESH,
    compiler_params=_SC_PARAMS,
    scratch_types=[
        pltpu.VMEM((N_PAD,), jnp.float32),
        pltpu.VMEM((N_PAD,), jnp.float32),
        pltpu.VMEM((N_PAD,), jnp.float32),
        pltpu.VMEM((2, SUP * CH), jnp.int32),
        pltpu.VMEM((SUP, 3, CH), jnp.int32),
    ],
)


def _make_sc_agg(C):
    def _scale(big_v, wring_v, ebuf_v, bg, g3):
        for j in range(CH // 16):
            wring_v[pl.ds(j * 16, 16)] = plsc.bitcast(
                ebuf_v[g3, 2, pl.ds(j * 16, 16)], jnp.float32)

        def grp(j2, carry):
            for l in range(16):
                e = j2 * 16 + l
                ws = plsc.load_gather(
                    wring_v, [jnp.full((16,), e, jnp.int32)])
                for k in range(C // 16):
                    big_v[bg, e, pl.ds(k * 16, 16)] = (
                        big_v[bg, e, pl.ds(k * 16, 16)] * ws)
            return carry

        lax.fori_loop(0, CH // 16, grp, 0)

    def body(hp_hbm, pk_hbm, s_out,
             ebuf_v, big_v, wring_v, s_sh, sem_g, sem_s, sem_i):
        c = lax.axis_index("c")
        s = lax.axis_index("s")
        wid = c * 16 + s
        cbase = wid * CPW
        r0 = s * ROWS_PT

        def pk_start(slot, g):
            pltpu.async_copy(pk_hbm.at[cbase + g], ebuf_v.at[slot], sem_i)

        def pk_wait(slot, g):
            pltpu.make_async_copy(
                pk_hbm.at[cbase + g], ebuf_v.at[slot], sem_i).wait()

        pltpu.sync_copy(pk_hbm.at[cbase], ebuf_v.at[0])
        pltpu.async_copy(hp_hbm.at[ebuf_v.at[0, 0]], big_v.at[0], sem_g)
        pk_start(1, 1)

        zv = jnp.zeros((16,), jnp.float32)

        def zrow(i, carry):
            for j in range(C // 16):
                big_v[1, i, pl.ds(j * 16, 16)] = zv
            return carry

        lax.fori_loop(0, CH, zrow, 0)
        for i in range(ROWS_PT // CH):
            pltpu.sync_copy(big_v.at[1], s_sh.at[pl.ds(r0 + i * CH, CH)])
        plsc.subcore_barrier()

        pltpu.make_async_copy(
            hp_hbm.at[ebuf_v.at[0, 0]], big_v.at[0], sem_g).wait()
        pk_start(2, 2)
        pk_wait(1, 1)
        pltpu.async_copy(hp_hbm.at[ebuf_v.at[1, 0]], big_v.at[1], sem_g)
        _scale(big_v, wring_v, ebuf_v, 0, 0)
        pltpu.async_copy(big_v.at[0], s_sh.at[ebuf_v.at[0, 1]], sem_s,
                         add=True)

        def iter_g(g, carry):
            bg = lax.rem(g, 2)
            nbg = 1 - bg
            g3 = lax.rem(g, 3)
            gn3 = lax.rem(g + 1, 3)
            gp3 = lax.rem(g + 2, 3)
            pltpu.make_async_copy(
                hp_hbm.at[ebuf_v.at[g3, 0]], big_v.at[bg], sem_g).wait()
            pltpu.make_async_copy(
                big_v.at[nbg], s_sh.at[ebuf_v.at[gp3, 1]], sem_s).wait()
            pk_start(gp3, jnp.minimum(g + 2, CPW - 1))
            pk_wait(gn3, jnp.minimum(g + 1, CPW - 1))
            pltpu.async_copy(
                hp_hbm.at[ebuf_v.at[gn3, 0]], big_v.at[nbg], sem_g)
            _scale(big_v, wring_v, ebuf_v, bg, g3)
            pltpu.async_copy(big_v.at[bg], s_sh.at[ebuf_v.at[g3, 1]], sem_s,
                             add=True)
            return carry

        lax.fori_loop(1, CPW, iter_g, 0)

        bl = (CPW - 1) % 2
        pltpu.make_async_copy(
            big_v.at[bl], s_sh.at[ebuf_v.at[(CPW - 1) % 3, 1]], sem_s).wait()
        pltpu.make_async_copy(
            hp_hbm.at[ebuf_v.at[CPW % 3, 0]], big_v.at[1 - bl], sem_g).wait()
        pk_wait((CPW + 1) % 3, CPW - 1)
        plsc.subcore_barrier()
        pltpu.sync_copy(s_sh.at[pl.ds(r0, ROWS_PT)],
                        s_out.at[pl.ds(c * N_PAD + r0, ROWS_PT)])

    return pl.kernel(
        body,
        out_type=jax.ShapeDtypeStruct((2 * N_PAD, C), jnp.float32),
        mesh=_SC_MESH,
        compiler_params=_SC_PARAMS,
        scratch_types=[
            pltpu.VMEM((3, 3, CH), jnp.int32),
            pltpu.VMEM((2, CH, C), jnp.float32),
            pltpu.VMEM((CH,), jnp.float32),
            pltpu.VMEM_SHARED((N_PAD, C), jnp.float32),
            pltpu.SemaphoreType.DMA,
            pltpu.SemaphoreType.DMA,
            pltpu.SemaphoreType.DMA,
        ],
    )


_sc_agg_1 = _make_sc_agg(D_HID)
_sc_agg_2 = _make_sc_agg(D_OUT)

_dense_a = pl.pallas_call(
    _tc_dense_a,
    out_shape=(
        jax.ShapeDtypeStruct((N_PAD, D_HID), jnp.float32),
        jax.ShapeDtypeStruct((N_PAD,), jnp.float32),
        jax.ShapeDtypeStruct((N_PAD,), jnp.float32),
    ),
)

_mid = pl.pallas_call(
    _tc_mid,
    out_shape=(
        jax.ShapeDtypeStruct((N_PAD, D_OUT), jnp.float32),
        jax.ShapeDtypeStruct((N_PAD,), jnp.float32),
        jax.ShapeDtypeStruct((N_PAD,), jnp.float32),
    ),
)

_final = pl.pallas_call(
    _tc_final,
    out_shape=jax.ShapeDtypeStruct((N_PAD, D_OUT), jnp.float32),
)


def kernel(x, edge_index, W1, att_src1, att_dst1, b1,
           W2, att_src2, att_dst2, b2):
    x = x.astype(jnp.float32)
    loops = jnp.arange(N_NODES, dtype=jnp.int32)
    padv = jnp.full((E_PAD - E_TOT,), N_NODES, dtype=jnp.int32)
    src = jnp.concatenate([edge_index[0].astype(jnp.int32), loops, padv])
    dst = jnp.concatenate([edge_index[1].astype(jnp.int32), loops, padv])

    x_pad = jnp.zeros((N_PAD, D_IN), jnp.float32).at[:N_NODES].set(x)

    h1, as1, ad1 = _dense_a(x_pad, W1, att_src1.reshape(1, D_HID),
                            att_dst1.reshape(1, D_HID))
    pk1, dn1 = _sc_w(src, dst, as1, ad1)
    s1 = _sc_agg_1(h1, pk1).reshape(2, N_PAD, D_HID)
    h2, as2, ad2 = _mid(s1, dn1, b1.reshape(1, D_HID), W2,
                        att_src2.reshape(1, D_OUT), att_dst2.reshape(1, D_OUT))
    pk2, dn2 = _sc_w(src, dst, as2, ad2)
    s2 = _sc_agg_2(h2, pk2).reshape(2, N_PAD, D_OUT)
    out = _final(s2, dn2, b2.reshape(1, D_OUT))
    return out[:N_NODES]
